# baseline (speedup 1.0000x reference)
import functools

import jax
import jax.numpy as jnp
from jax import lax
from jax.experimental import pallas as pl
from jax.experimental.pallas import tpu as pltpu
from jax.experimental.pallas import tpu_sc as plsc

E = 16
K = 2
NT = 4096
D = 1024
H = 2048
T = 128
P = NT * K + E * T
NB = P // T
CHUNK = 512
NC, NS = 2, 16
NW = NC * NS
TPW = NT // NW
SUB = 32
NSUB = TPW // SUB


def _route_body(x_ref, wg_ref, g0_ref, g1_ref, e0_ref, e1_ref,
                r0_ref, r1_ref, cnt_ref, run_ref):
    step = pl.program_id(0)

    @pl.when(step == 0)
    def _():
        run_ref[...] = jnp.zeros_like(run_ref)

    logits = jnp.dot(x_ref[...], wg_ref[...],
                     preferred_element_type=jnp.float32)
    m = jnp.max(logits, axis=-1, keepdims=True)
    ex = jnp.exp(logits - m)
    probs = ex / jnp.sum(ex, axis=-1, keepdims=True)

    iota_e = lax.broadcasted_iota(jnp.int32, (CHUNK, E), 1)
    m1 = jnp.max(probs, axis=-1, keepdims=True)
    a1 = jnp.min(jnp.where(probs == m1, iota_e, E), axis=-1, keepdims=True)
    probs2 = jnp.where(iota_e == a1, -1.0, probs)
    m2 = jnp.max(probs2, axis=-1, keepdims=True)
    a2 = jnp.min(jnp.where(probs2 == m2, iota_e, E), axis=-1, keepdims=True)

    g0_ref[...] = m1
    g1_ref[...] = m2
    e0_ref[...] = a1
    e1_ref[...] = a2

    oh1 = (iota_e == a1).astype(jnp.float32)
    oh2 = (iota_e == a2).astype(jnp.float32)
    oh = jnp.concatenate([oh1, oh2], axis=0)

    n2 = 2 * CHUNK
    rr = lax.broadcasted_iota(jnp.int32, (n2, n2), 0)
    cc = lax.broadcasted_iota(jnp.int32, (n2, n2), 1)
    lt = (rr > cc).astype(jnp.float32)
    ranks = jnp.dot(lt, oh, preferred_element_type=jnp.float32)
    ranks = ranks + run_ref[0:1, :]
    grank = jnp.sum(ranks * oh, axis=-1, keepdims=True)
    r0_ref[...] = grank[:CHUNK, :].astype(jnp.int32)
    r1_ref[...] = grank[CHUNK:, :].astype(jnp.int32)

    run_ref[0:1, :] = run_ref[0:1, :] + jnp.sum(oh, axis=0, keepdims=True)
    cnt_ref[...] = run_ref[...]


def _route(xf, w_gate):
    n_steps = NT // CHUNK
    col_i = jax.ShapeDtypeStruct((NT, 1), jnp.int32)
    col_f = jax.ShapeDtypeStruct((NT, 1), jnp.float32)
    spec_col = pl.BlockSpec((CHUNK, 1), lambda i: (i, 0))
    return pl.pallas_call(
        _route_body,
        grid=(n_steps,),
        in_specs=[
            pl.BlockSpec((CHUNK, D), lambda i: (i, 0)),
            pl.BlockSpec((D, E), lambda i: (0, 0)),
        ],
        out_specs=[spec_col, spec_col, spec_col, spec_col, spec_col, spec_col,
                   pl.BlockSpec((8, E), lambda i: (0, 0))],
        out_shape=[col_f, col_f, col_i, col_i, col_i, col_i,
                   jax.ShapeDtypeStruct((8, E), jnp.float32)],
        scratch_shapes=[pltpu.VMEM((8, E), jnp.float32)],
    )(xf, w_gate)


def _shift_lanes(a, k):
    return jnp.concatenate([jnp.zeros((1, k), a.dtype), a[:, :E - k]], axis=1)


def _finalize_body(cnt_ref, e0_ref, e1_ref, r0_ref, r1_ref,
                   p0_ref, p1_ref, be_ref, bv_ref):
    c = cnt_ref[0:1, :].astype(jnp.int32)
    padded = ((c + T - 1) // T) * T
    s = padded
    for k in (1, 2, 4, 8):
        s = s + _shift_lanes(s, k)
    offs = _shift_lanes(s, 1)
    total = jnp.sum(padded)

    def lookup(e_col):
        cmp = e_col == lax.broadcasted_iota(jnp.int32, (NT, E), 1)
        return jnp.sum(jnp.where(cmp, offs, 0), axis=1, keepdims=True)

    p0_ref[...] = r0_ref[...] + lookup(e0_ref[...])
    p1_ref[...] = r1_ref[...] + lookup(e1_ref[...])

    bT = lax.broadcasted_iota(jnp.int32, (NB, E), 0) * T
    be = jnp.sum((offs <= bT).astype(jnp.int32), axis=1, keepdims=True) - 1
    be_ref[...] = jnp.clip(be, 0, E - 1)
    bv_ref[...] = (lax.broadcasted_iota(jnp.int32, (NB, 1), 0) * T
                   < total).astype(jnp.int32)


def _finalize(cnt, e0, e1, r0, r1):
    col_i = jax.ShapeDtypeStruct((NT, 1), jnp.int32)
    blk_i = jax.ShapeDtypeStruct((NB, 1), jnp.int32)
    return pl.pallas_call(
        _finalize_body,
        out_shape=[col_i, col_i, blk_i, blk_i],
    )(cnt, e0, e1, r0, r1)


def _dispatch(xf, pos0, pos1):
    mesh = plsc.VectorSubcoreMesh(core_axis_name="c", subcore_axis_name="s")

    @functools.partial(
        pl.kernel,
        mesh=mesh,
        out_type=jax.ShapeDtypeStruct((P, D), jnp.float32),
        scratch_types=[
            pltpu.VMEM((SUB, D), jnp.float32),
            pltpu.VMEM((SUB,), jnp.int32),
            pltpu.VMEM((SUB,), jnp.int32),
            pltpu.SemaphoreType.DMA,
        ],
    )
    def k(x_hbm, p0_hbm, p1_hbm, out_hbm, rows_v, i0_v, i1_v, sem):
        wid = lax.axis_index("s") * NC + lax.axis_index("c")

        def sub(j, carry):
            t0 = wid * TPW + j * SUB
            pltpu.sync_copy(x_hbm.at[pl.ds(t0, SUB)], rows_v)
            pltpu.sync_copy(p0_hbm.at[pl.ds(t0, SUB)], i0_v)
            pltpu.sync_copy(p1_hbm.at[pl.ds(t0, SUB)], i1_v)
            pltpu.async_copy(rows_v, out_hbm.at[i0_v], sem).wait()
            pltpu.async_copy(rows_v, out_hbm.at[i1_v], sem).wait()
            return carry

        lax.fori_loop(0, NSUB, sub, 0)

    return k(xf, pos0, pos1)


def _ffn_body(be_ref, bv_ref, x_ref, w1_ref, w2_ref, o_ref):
    b = pl.program_id(0)

    @pl.when(bv_ref[b] != 0)
    def _():
        xb = x_ref[...].astype(jnp.bfloat16)
        h = jnp.dot(xb, w1_ref[0], preferred_element_type=jnp.float32)
        h = jax.nn.gelu(h).astype(jnp.bfloat16)
        o_ref[...] = jnp.dot(h, w2_ref[0], preferred_element_type=jnp.float32)

    @pl.when(bv_ref[b] == 0)
    def _():
        o_ref[...] = jnp.zeros_like(o_ref)


def _ffn(bexp, bval, ei, W1, W2):
    grid_spec = pltpu.PrefetchScalarGridSpec(
        num_scalar_prefetch=2,
        grid=(NB,),
        in_specs=[
            pl.BlockSpec((T, D), lambda b, be, bv: (b, 0)),
            pl.BlockSpec((1, D, H), lambda b, be, bv: (be[b], 0, 0)),
            pl.BlockSpec((1, H, D), lambda b, be, bv: (be[b], 0, 0)),
        ],
        out_specs=pl.BlockSpec((T, D), lambda b, be, bv: (b, 0)),
    )
    return pl.pallas_call(
        _ffn_body,
        grid_spec=grid_spec,
        out_shape=jax.ShapeDtypeStruct((P, D), jnp.float32),
    )(bexp, bval, ei, W1, W2)


def _combine_gather(outs, pos0, pos1):
    mesh = plsc.VectorSubcoreMesh(core_axis_name="c", subcore_axis_name="s")

    @functools.partial(
        pl.kernel,
        mesh=mesh,
        out_type=[jax.ShapeDtypeStruct((NT, D), jnp.float32),
                  jax.ShapeDtypeStruct((NT, D), jnp.float32)],
        scratch_types=[
            pltpu.VMEM((SUB, D), jnp.float32),
            pltpu.VMEM((SUB, D), jnp.float32),
            pltpu.VMEM((SUB,), jnp.int32),
            pltpu.VMEM((SUB,), jnp.int32),
            pltpu.SemaphoreType.DMA,
        ],
    )
    def k(os_hbm, p0_hbm, p1_hbm, r0_hbm, r1_hbm,
          a_v, b_v, i0_v, i1_v, sem):
        wid = lax.axis_index("s") * NC + lax.axis_index("c")

        def sub(j, carry):
            t0 = wid * TPW + j * SUB
            pltpu.sync_copy(p0_hbm.at[pl.ds(t0, SUB)], i0_v)
            pltpu.sync_copy(p1_hbm.at[pl.ds(t0, SUB)], i1_v)
            pltpu.async_copy(os_hbm.at[i0_v], a_v, sem).wait()
            pltpu.async_copy(os_hbm.at[i1_v], b_v, sem).wait()
            pltpu.sync_copy(a_v, r0_hbm.at[pl.ds(t0, SUB)])
            pltpu.sync_copy(b_v, r1_hbm.at[pl.ds(t0, SUB)])
            return carry

        lax.fori_loop(0, NSUB, sub, 0)

    return k(outs, pos0, pos1)


def _wsum_body(r0_ref, r1_ref, g0_ref, g1_ref, y_ref):
    y_ref[...] = g0_ref[...] * r0_ref[...] + g1_ref[...] * r1_ref[...]


def _wsum(r0, r1, g0, g1):
    spec_row = pl.BlockSpec((CHUNK, D), lambda i: (i, 0))
    spec_col = pl.BlockSpec((CHUNK, 1), lambda i: (i, 0))
    return pl.pallas_call(
        _wsum_body,
        grid=(NT // CHUNK,),
        in_specs=[spec_row, spec_row, spec_col, spec_col],
        out_specs=spec_row,
        out_shape=jax.ShapeDtypeStruct((NT, D), jnp.float32),
    )(r0, r1, g0, g1)


def kernel(x, w_gate, W1, W2):
    bsz, length, d = x.shape
    xf = x.reshape(-1, d)
    g0, g1, e0, e1, r0, r1, cnt = _route(xf, w_gate)
    pos0, pos1, bexp, bval = _finalize(cnt, e0, e1, r0, r1)
    pos0 = pos0.reshape(-1)
    pos1 = pos1.reshape(-1)
    ei = _dispatch(xf, pos0, pos1)
    outs = _ffn(bexp.reshape(-1), bval.reshape(-1), ei,
                W1.astype(jnp.bfloat16), W2.astype(jnp.bfloat16))
    r0c, r1c = _combine_gather(outs, pos0, pos1)
    y = _wsum(r0c, r1c, g0, g1)
    return y.reshape(bsz, length, d)


# speedup vs baseline: 1.3137x; 1.3137x over previous
import functools

import jax
import jax.numpy as jnp
from jax import lax
from jax.experimental import pallas as pl
from jax.experimental.pallas import tpu as pltpu
from jax.experimental.pallas import tpu_sc as plsc

E = 16
K = 2
NT = 4096
D = 1024
H = 2048
T = 128
P = NT * K + E * T
NB = P // T
CHUNK = 512
NC, NS = 2, 16
NW = NC * NS
TPW = NT // NW
SUB = 32
NSUB = TPW // SUB


def _route_body(x_ref, wg_ref, g0_ref, g1_ref, e0_ref, e1_ref,
                r0_ref, r1_ref, cnt_ref, run_ref):
    step = pl.program_id(0)

    @pl.when(step == 0)
    def _():
        run_ref[...] = jnp.zeros_like(run_ref)

    logits = jnp.dot(x_ref[...], wg_ref[...],
                     preferred_element_type=jnp.float32)
    m = jnp.max(logits, axis=-1, keepdims=True)
    ex = jnp.exp(logits - m)
    probs = ex / jnp.sum(ex, axis=-1, keepdims=True)

    iota_e = lax.broadcasted_iota(jnp.int32, (CHUNK, E), 1)
    m1 = jnp.max(probs, axis=-1, keepdims=True)
    a1 = jnp.min(jnp.where(probs == m1, iota_e, E), axis=-1, keepdims=True)
    probs2 = jnp.where(iota_e == a1, -1.0, probs)
    m2 = jnp.max(probs2, axis=-1, keepdims=True)
    a2 = jnp.min(jnp.where(probs2 == m2, iota_e, E), axis=-1, keepdims=True)

    g0_ref[...] = m1
    g1_ref[...] = m2
    e0_ref[...] = a1
    e1_ref[...] = a2

    oh1 = (iota_e == a1).astype(jnp.float32)
    oh2 = (iota_e == a2).astype(jnp.float32)
    oh = jnp.concatenate([oh1, oh2], axis=0)

    n2 = 2 * CHUNK
    rr = lax.broadcasted_iota(jnp.int32, (n2, n2), 0)
    cc = lax.broadcasted_iota(jnp.int32, (n2, n2), 1)
    lt = (rr > cc).astype(jnp.float32)
    ranks = jnp.dot(lt, oh, preferred_element_type=jnp.float32)
    ranks = ranks + run_ref[0:1, :]
    grank = jnp.sum(ranks * oh, axis=-1, keepdims=True)
    r0_ref[...] = grank[:CHUNK, :].astype(jnp.int32)
    r1_ref[...] = grank[CHUNK:, :].astype(jnp.int32)

    run_ref[0:1, :] = run_ref[0:1, :] + jnp.sum(oh, axis=0, keepdims=True)
    cnt_ref[...] = run_ref[...]


def _route(xf, w_gate):
    n_steps = NT // CHUNK
    col_i = jax.ShapeDtypeStruct((NT, 1), jnp.int32)
    col_f = jax.ShapeDtypeStruct((NT, 1), jnp.float32)
    spec_col = pl.BlockSpec((CHUNK, 1), lambda i: (i, 0))
    return pl.pallas_call(
        _route_body,
        grid=(n_steps,),
        in_specs=[
            pl.BlockSpec((CHUNK, D), lambda i: (i, 0)),
            pl.BlockSpec((D, E), lambda i: (0, 0)),
        ],
        out_specs=[spec_col, spec_col, spec_col, spec_col, spec_col, spec_col,
                   pl.BlockSpec((8, E), lambda i: (0, 0))],
        out_shape=[col_f, col_f, col_i, col_i, col_i, col_i,
                   jax.ShapeDtypeStruct((8, E), jnp.float32)],
        scratch_shapes=[pltpu.VMEM((8, E), jnp.float32)],
    )(xf, w_gate)


def _shift_lanes(a, k):
    return jnp.concatenate([jnp.zeros((1, k), a.dtype), a[:, :E - k]], axis=1)


def _shift_rows(a, k):
    return jnp.concatenate([jnp.zeros((k, 1), a.dtype), a[:a.shape[0] - k, :]],
                           axis=0)


def _finalize_body(cnt_ref, e0_ref, e1_ref, r0_ref, r1_ref,
                   p0_ref, p1_ref, be_ref, bv_ref, sf_ref, so_ref,
                   nx_ref, ns_ref):
    c = cnt_ref[0:1, :].astype(jnp.int32)
    padded = ((c + T - 1) // T) * T
    s = padded
    for k in (1, 2, 4, 8):
        s = s + _shift_lanes(s, k)
    offs = _shift_lanes(s, 1)
    total = jnp.sum(padded)

    def lookup(e_col):
        cmp = e_col == lax.broadcasted_iota(jnp.int32, (NT, E), 1)
        return jnp.sum(jnp.where(cmp, offs, 0), axis=1, keepdims=True)

    p0_ref[...] = r0_ref[...] + lookup(e0_ref[...])
    p1_ref[...] = r1_ref[...] + lookup(e1_ref[...])

    iota_be = lax.broadcasted_iota(jnp.int32, (NB, E), 1)
    bT = lax.broadcasted_iota(jnp.int32, (NB, E), 0) * T
    be = jnp.sum((offs <= bT).astype(jnp.int32), axis=1, keepdims=True) - 1
    be = jnp.clip(be, 0, E - 1)
    bv = (lax.broadcasted_iota(jnp.int32, (NB, 1), 0) * T
          < total).astype(jnp.int32)
    be_ref[...] = be
    bv_ref[...] = bv

    prev = jnp.concatenate([jnp.full((1, 1), -1, jnp.int32), be[:-1, :]],
                           axis=0)
    sf = ((be != prev) & (bv != 0)).astype(jnp.int32)
    sf_ref[...] = sf
    cum = sf
    for k in (1, 2, 4, 8, 16, 32, 64):
        if k < NB:
            cum = cum + _shift_rows(cum, k)
    so_ref[...] = cum - 1
    ns_ref[...] = jnp.sum(sf) * jnp.ones((1, 1), jnp.int32)

    cand = (iota_be > be) & (padded > 0)
    nx_ref[...] = jnp.min(jnp.where(cand, iota_be, E - 1), axis=1,
                          keepdims=True)


def _finalize(cnt, e0, e1, r0, r1):
    col_i = jax.ShapeDtypeStruct((NT, 1), jnp.int32)
    blk_i = jax.ShapeDtypeStruct((NB, 1), jnp.int32)
    one_i = jax.ShapeDtypeStruct((1, 1), jnp.int32)
    return pl.pallas_call(
        _finalize_body,
        out_shape=[col_i, col_i, blk_i, blk_i, blk_i, blk_i, blk_i, one_i],
    )(cnt, e0, e1, r0, r1)


def _dispatch(xf, pos0, pos1):
    mesh = plsc.VectorSubcoreMesh(core_axis_name="c", subcore_axis_name="s")

    @functools.partial(
        pl.kernel,
        mesh=mesh,
        out_type=jax.ShapeDtypeStruct((P, D), jnp.float32),
        scratch_types=[
            pltpu.VMEM((SUB, D), jnp.float32),
            pltpu.VMEM((SUB,), jnp.int32),
            pltpu.VMEM((SUB,), jnp.int32),
            pltpu.SemaphoreType.DMA,
        ],
    )
    def k(x_hbm, p0_hbm, p1_hbm, out_hbm, rows_v, i0_v, i1_v, sem):
        wid = lax.axis_index("s") * NC + lax.axis_index("c")

        def sub(j, carry):
            t0 = wid * TPW + j * SUB
            pltpu.sync_copy(x_hbm.at[pl.ds(t0, SUB)], rows_v)
            pltpu.sync_copy(p0_hbm.at[pl.ds(t0, SUB)], i0_v)
            pltpu.sync_copy(p1_hbm.at[pl.ds(t0, SUB)], i1_v)
            pltpu.async_copy(rows_v, out_hbm.at[i0_v], sem).wait()
            pltpu.async_copy(rows_v, out_hbm.at[i1_v], sem).wait()
            return carry

        lax.fori_loop(0, NSUB, sub, 0)

    return k(xf, pos0, pos1)


def _ffn_body(be_ref, bv_ref, sf_ref, so_ref, nx_ref, ns_ref,
              x_ref, w1_hbm, w2_hbm, o_ref, w1b, w2b, s1, s2):
    b = pl.program_id(0)
    s = so_ref[b]
    slot = lax.rem(s, 2)

    @pl.when(b == 0)
    def _():
        pltpu.make_async_copy(w1_hbm.at[be_ref[0]], w1b.at[0], s1.at[0]).start()
        pltpu.make_async_copy(w2_hbm.at[be_ref[0]], w2b.at[0], s2.at[0]).start()

    @pl.when(sf_ref[b] != 0)
    def _():
        pltpu.make_async_copy(w1_hbm.at[be_ref[b]], w1b.at[slot],
                              s1.at[slot]).wait()
        pltpu.make_async_copy(w2_hbm.at[be_ref[b]], w2b.at[slot],
                              s2.at[slot]).wait()

        @pl.when(s + 1 < ns_ref[0])
        def _():
            nslot = 1 - slot
            pltpu.make_async_copy(w1_hbm.at[nx_ref[b]], w1b.at[nslot],
                                  s1.at[nslot]).start()
            pltpu.make_async_copy(w2_hbm.at[nx_ref[b]], w2b.at[nslot],
                                  s2.at[nslot]).start()

    @pl.when(bv_ref[b] != 0)
    def _():
        h = jnp.dot(x_ref[...], w1b[slot], preferred_element_type=jnp.float32)
        h = jax.nn.gelu(h)
        o_ref[...] = jnp.dot(h, w2b[slot], preferred_element_type=jnp.float32)

    @pl.when(bv_ref[b] == 0)
    def _():
        o_ref[...] = jnp.zeros_like(o_ref)


def _ffn(bexp, bval, sf, so, nx, ns, ei, W1, W2):
    grid_spec = pltpu.PrefetchScalarGridSpec(
        num_scalar_prefetch=6,
        grid=(NB,),
        in_specs=[
            pl.BlockSpec((T, D), lambda b, *_: (b, 0)),
            pl.BlockSpec(memory_space=pl.ANY),
            pl.BlockSpec(memory_space=pl.ANY),
        ],
        out_specs=pl.BlockSpec((T, D), lambda b, *_: (b, 0)),
        scratch_shapes=[
            pltpu.VMEM((2, D, H), jnp.float32),
            pltpu.VMEM((2, H, D), jnp.float32),
            pltpu.SemaphoreType.DMA((2,)),
            pltpu.SemaphoreType.DMA((2,)),
        ],
    )
    return pl.pallas_call(
        _ffn_body,
        grid_spec=grid_spec,
        out_shape=jax.ShapeDtypeStruct((P, D), jnp.float32),
    )(bexp, bval, sf, so, nx, ns, ei, W1, W2)


def _combine_gather(outs, pos0, pos1):
    mesh = plsc.VectorSubcoreMesh(core_axis_name="c", subcore_axis_name="s")

    @functools.partial(
        pl.kernel,
        mesh=mesh,
        out_type=[jax.ShapeDtypeStruct((NT, D), jnp.float32),
                  jax.ShapeDtypeStruct((NT, D), jnp.float32)],
        scratch_types=[
            pltpu.VMEM((SUB, D), jnp.float32),
            pltpu.VMEM((SUB, D), jnp.float32),
            pltpu.VMEM((SUB,), jnp.int32),
            pltpu.VMEM((SUB,), jnp.int32),
            pltpu.SemaphoreType.DMA,
        ],
    )
    def k(os_hbm, p0_hbm, p1_hbm, r0_hbm, r1_hbm,
          a_v, b_v, i0_v, i1_v, sem):
        wid = lax.axis_index("s") * NC + lax.axis_index("c")

        def sub(j, carry):
            t0 = wid * TPW + j * SUB
            pltpu.sync_copy(p0_hbm.at[pl.ds(t0, SUB)], i0_v)
            pltpu.sync_copy(p1_hbm.at[pl.ds(t0, SUB)], i1_v)
            pltpu.async_copy(os_hbm.at[i0_v], a_v, sem).wait()
            pltpu.async_copy(os_hbm.at[i1_v], b_v, sem).wait()
            pltpu.sync_copy(a_v, r0_hbm.at[pl.ds(t0, SUB)])
            pltpu.sync_copy(b_v, r1_hbm.at[pl.ds(t0, SUB)])
            return carry

        lax.fori_loop(0, NSUB, sub, 0)

    return k(outs, pos0, pos1)


def _wsum_body(r0_ref, r1_ref, g0_ref, g1_ref, y_ref):
    y_ref[...] = g0_ref[...] * r0_ref[...] + g1_ref[...] * r1_ref[...]


def _wsum(r0, r1, g0, g1):
    spec_row = pl.BlockSpec((CHUNK, D), lambda i: (i, 0))
    spec_col = pl.BlockSpec((CHUNK, 1), lambda i: (i, 0))
    return pl.pallas_call(
        _wsum_body,
        grid=(NT // CHUNK,),
        in_specs=[spec_row, spec_row, spec_col, spec_col],
        out_specs=spec_row,
        out_shape=jax.ShapeDtypeStruct((NT, D), jnp.float32),
    )(r0, r1, g0, g1)


def kernel(x, w_gate, W1, W2):
    bsz, length, d = x.shape
    xf = x.reshape(-1, d)
    g0, g1, e0, e1, r0, r1, cnt = _route(xf, w_gate)
    pos0, pos1, bexp, bval, sf, so, nx, ns = _finalize(cnt, e0, e1, r0, r1)
    pos0 = pos0.reshape(-1)
    pos1 = pos1.reshape(-1)
    ei = _dispatch(xf, pos0, pos1)
    outs = _ffn(bexp.reshape(-1), bval.reshape(-1), sf.reshape(-1),
                so.reshape(-1), nx.reshape(-1), ns.reshape(-1), ei, W1, W2)
    r0c, r1c = _combine_gather(outs, pos0, pos1)
    y = _wsum(r0c, r1c, g0, g1)
    return y.reshape(bsz, length, d)


# speedup vs baseline: 1.3294x; 1.0119x over previous
import functools

import jax
import jax.numpy as jnp
from jax import lax
from jax.experimental import pallas as pl
from jax.experimental.pallas import tpu as pltpu
from jax.experimental.pallas import tpu_sc as plsc

E = 16
K = 2
NT = 4096
D = 1024
H = 2048
T = 128
P = NT * K + E * T
NB = P // T
CHUNK = 512
NC, NS = 2, 16
NW = NC * NS
TPW = NT // NW
SUB = 32
NSUB = TPW // SUB


def _route_body(x_ref, wg_ref, g0_ref, g1_ref, e0_ref, e1_ref,
                r0_ref, r1_ref, cnt_ref, gs0_ref, gs1_ref, run_ref):
    step = pl.program_id(0)

    @pl.when(step == 0)
    def _():
        run_ref[...] = jnp.zeros_like(run_ref)

    logits = jnp.dot(x_ref[...], wg_ref[...],
                     preferred_element_type=jnp.float32)
    m = jnp.max(logits, axis=-1, keepdims=True)
    ex = jnp.exp(logits - m)
    probs = ex / jnp.sum(ex, axis=-1, keepdims=True)

    iota_e = lax.broadcasted_iota(jnp.int32, (CHUNK, E), 1)
    m1 = jnp.max(probs, axis=-1, keepdims=True)
    a1 = jnp.min(jnp.where(probs == m1, iota_e, E), axis=-1, keepdims=True)
    probs2 = jnp.where(iota_e == a1, -1.0, probs)
    m2 = jnp.max(probs2, axis=-1, keepdims=True)
    a2 = jnp.min(jnp.where(probs2 == m2, iota_e, E), axis=-1, keepdims=True)

    g0_ref[...] = m1
    g1_ref[...] = m2
    e0_ref[...] = a1
    e1_ref[...] = a2
    gs0_ref[...] = jnp.broadcast_to(m1, (CHUNK, E))
    gs1_ref[...] = jnp.broadcast_to(m2, (CHUNK, E))

    oh1 = (iota_e == a1).astype(jnp.float32)
    oh2 = (iota_e == a2).astype(jnp.float32)
    oh = jnp.concatenate([oh1, oh2], axis=0)

    n2 = 2 * CHUNK
    rr = lax.broadcasted_iota(jnp.int32, (n2, n2), 0)
    cc = lax.broadcasted_iota(jnp.int32, (n2, n2), 1)
    lt = (rr > cc).astype(jnp.float32)
    ranks = jnp.dot(lt, oh, preferred_element_type=jnp.float32)
    ranks = ranks + run_ref[0:1, :]
    grank = jnp.sum(ranks * oh, axis=-1, keepdims=True)
    r0_ref[...] = grank[:CHUNK, :].astype(jnp.int32)
    r1_ref[...] = grank[CHUNK:, :].astype(jnp.int32)

    run_ref[0:1, :] = run_ref[0:1, :] + jnp.sum(oh, axis=0, keepdims=True)
    cnt_ref[...] = run_ref[...]


def _route(xf, w_gate):
    n_steps = NT // CHUNK
    col_i = jax.ShapeDtypeStruct((NT, 1), jnp.int32)
    col_f = jax.ShapeDtypeStruct((NT, 1), jnp.float32)
    spec_col = pl.BlockSpec((CHUNK, 1), lambda i: (i, 0))
    return pl.pallas_call(
        _route_body,
        grid=(n_steps,),
        in_specs=[
            pl.BlockSpec((CHUNK, D), lambda i: (i, 0)),
            pl.BlockSpec((D, E), lambda i: (0, 0)),
        ],
        out_specs=[spec_col, spec_col, spec_col, spec_col, spec_col, spec_col,
                   pl.BlockSpec((8, E), lambda i: (0, 0)),
                   pl.BlockSpec((CHUNK, E), lambda i: (i, 0)),
                   pl.BlockSpec((CHUNK, E), lambda i: (i, 0))],
        out_shape=[col_f, col_f, col_i, col_i, col_i, col_i,
                   jax.ShapeDtypeStruct((8, E), jnp.float32),
                   jax.ShapeDtypeStruct((NT, E), jnp.float32),
                   jax.ShapeDtypeStruct((NT, E), jnp.float32)],
        scratch_shapes=[pltpu.VMEM((8, E), jnp.float32)],
    )(xf, w_gate)


def _shift_lanes(a, k):
    return jnp.concatenate([jnp.zeros((1, k), a.dtype), a[:, :E - k]], axis=1)


def _shift_rows(a, k):
    return jnp.concatenate([jnp.zeros((k, 1), a.dtype), a[:a.shape[0] - k, :]],
                           axis=0)


def _finalize_body(cnt_ref, e0_ref, e1_ref, r0_ref, r1_ref,
                   p0_ref, p1_ref, be_ref, bv_ref, sf_ref, so_ref,
                   nx_ref, ns_ref):
    c = cnt_ref[0:1, :].astype(jnp.int32)
    padded = ((c + T - 1) // T) * T
    s = padded
    for k in (1, 2, 4, 8):
        s = s + _shift_lanes(s, k)
    offs = _shift_lanes(s, 1)
    total = jnp.sum(padded)

    def lookup(e_col):
        cmp = e_col == lax.broadcasted_iota(jnp.int32, (NT, E), 1)
        return jnp.sum(jnp.where(cmp, offs, 0), axis=1, keepdims=True)

    p0_ref[...] = r0_ref[...] + lookup(e0_ref[...])
    p1_ref[...] = r1_ref[...] + lookup(e1_ref[...])

    iota_be = lax.broadcasted_iota(jnp.int32, (NB, E), 1)
    bT = lax.broadcasted_iota(jnp.int32, (NB, E), 0) * T
    be = jnp.sum((offs <= bT).astype(jnp.int32), axis=1, keepdims=True) - 1
    be = jnp.clip(be, 0, E - 1)
    bv = (lax.broadcasted_iota(jnp.int32, (NB, 1), 0) * T
          < total).astype(jnp.int32)
    be_ref[...] = be
    bv_ref[...] = bv

    prev = jnp.concatenate([jnp.full((1, 1), -1, jnp.int32), be[:-1, :]],
                           axis=0)
    sf = ((be != prev) & (bv != 0)).astype(jnp.int32)
    sf_ref[...] = sf
    cum = sf
    for k in (1, 2, 4, 8, 16, 32, 64):
        if k < NB:
            cum = cum + _shift_rows(cum, k)
    so_ref[...] = cum - 1
    ns_ref[...] = jnp.sum(sf) * jnp.ones((1, 1), jnp.int32)

    cand = (iota_be > be) & (padded > 0)
    nx_ref[...] = jnp.min(jnp.where(cand, iota_be, E - 1), axis=1,
                          keepdims=True)


def _finalize(cnt, e0, e1, r0, r1):
    col_i = jax.ShapeDtypeStruct((NT, 1), jnp.int32)
    blk_i = jax.ShapeDtypeStruct((NB, 1), jnp.int32)
    one_i = jax.ShapeDtypeStruct((1, 1), jnp.int32)
    return pl.pallas_call(
        _finalize_body,
        out_shape=[col_i, col_i, blk_i, blk_i, blk_i, blk_i, blk_i, one_i],
    )(cnt, e0, e1, r0, r1)


def _dispatch(xf, pos0, pos1):
    mesh = plsc.VectorSubcoreMesh(core_axis_name="c", subcore_axis_name="s")

    @functools.partial(
        pl.kernel,
        mesh=mesh,
        out_type=jax.ShapeDtypeStruct((P, D), jnp.float32),
        scratch_types=[
            pltpu.VMEM((SUB, D), jnp.float32),
            pltpu.VMEM((SUB,), jnp.int32),
            pltpu.VMEM((SUB,), jnp.int32),
            pltpu.SemaphoreType.DMA,
        ],
    )
    def k(x_hbm, p0_hbm, p1_hbm, out_hbm, rows_v, i0_v, i1_v, sem):
        wid = lax.axis_index("s") * NC + lax.axis_index("c")

        def sub(j, carry):
            t0 = wid * TPW + j * SUB
            pltpu.sync_copy(x_hbm.at[pl.ds(t0, SUB)], rows_v)
            pltpu.sync_copy(p0_hbm.at[pl.ds(t0, SUB)], i0_v)
            pltpu.sync_copy(p1_hbm.at[pl.ds(t0, SUB)], i1_v)
            pltpu.async_copy(rows_v, out_hbm.at[i0_v], sem).wait()
            pltpu.async_copy(rows_v, out_hbm.at[i1_v], sem).wait()
            return carry

        lax.fori_loop(0, NSUB, sub, 0)

    return k(xf, pos0, pos1)


def _ffn_body(be_ref, bv_ref, sf_ref, so_ref, nx_ref, ns_ref,
              x_ref, w1_hbm, w2_hbm, o_ref, w1b, w2b, s1, s2):
    b = pl.program_id(0)
    s = so_ref[b]
    slot = lax.rem(s, 2)

    @pl.when(b == 0)
    def _():
        pltpu.make_async_copy(w1_hbm.at[be_ref[0]], w1b.at[0], s1.at[0]).start()
        pltpu.make_async_copy(w2_hbm.at[be_ref[0]], w2b.at[0], s2.at[0]).start()

    @pl.when(sf_ref[b] != 0)
    def _():
        pltpu.make_async_copy(w1_hbm.at[be_ref[b]], w1b.at[slot],
                              s1.at[slot]).wait()
        pltpu.make_async_copy(w2_hbm.at[be_ref[b]], w2b.at[slot],
                              s2.at[slot]).wait()

        @pl.when(s + 1 < ns_ref[0])
        def _():
            nslot = 1 - slot
            pltpu.make_async_copy(w1_hbm.at[nx_ref[b]], w1b.at[nslot],
                                  s1.at[nslot]).start()
            pltpu.make_async_copy(w2_hbm.at[nx_ref[b]], w2b.at[nslot],
                                  s2.at[nslot]).start()

    @pl.when(bv_ref[b] != 0)
    def _():
        h = jnp.dot(x_ref[...], w1b[slot], preferred_element_type=jnp.float32)
        h = jax.nn.gelu(h)
        o_ref[...] = jnp.dot(h, w2b[slot], preferred_element_type=jnp.float32)

    @pl.when(bv_ref[b] == 0)
    def _():
        o_ref[...] = jnp.zeros_like(o_ref)


def _ffn(bexp, bval, sf, so, nx, ns, ei, W1, W2):
    grid_spec = pltpu.PrefetchScalarGridSpec(
        num_scalar_prefetch=6,
        grid=(NB,),
        in_specs=[
            pl.BlockSpec((T, D), lambda b, *_: (b, 0)),
            pl.BlockSpec(memory_space=pl.ANY),
            pl.BlockSpec(memory_space=pl.ANY),
        ],
        out_specs=pl.BlockSpec((T, D), lambda b, *_: (b, 0)),
        scratch_shapes=[
            pltpu.VMEM((2, D, H), jnp.float32),
            pltpu.VMEM((2, H, D), jnp.float32),
            pltpu.SemaphoreType.DMA((2,)),
            pltpu.SemaphoreType.DMA((2,)),
        ],
    )
    return pl.pallas_call(
        _ffn_body,
        grid_spec=grid_spec,
        out_shape=jax.ShapeDtypeStruct((P, D), jnp.float32),
    )(bexp, bval, sf, so, nx, ns, ei, W1, W2)


def _combine(outs, pos0, pos1, gs0, gs1):
    mesh = plsc.VectorSubcoreMesh(core_axis_name="c", subcore_axis_name="s")

    @functools.partial(
        pl.kernel,
        mesh=mesh,
        out_type=jax.ShapeDtypeStruct((NT, D), jnp.float32),
        scratch_types=[
            pltpu.VMEM((SUB, D), jnp.float32),
            pltpu.VMEM((SUB, D), jnp.float32),
            pltpu.VMEM((SUB, E), jnp.float32),
            pltpu.VMEM((SUB, E), jnp.float32),
            pltpu.VMEM((SUB,), jnp.int32),
            pltpu.VMEM((SUB,), jnp.int32),
            pltpu.SemaphoreType.DMA,
        ],
    )
    def k(os_hbm, p0_hbm, p1_hbm, gs0_hbm, gs1_hbm, y_hbm,
          a_v, b_v, ga_v, gb_v, i0_v, i1_v, sem):
        wid = lax.axis_index("s") * NC + lax.axis_index("c")

        def sub(j, carry):
            t0 = wid * TPW + j * SUB
            pltpu.sync_copy(p0_hbm.at[pl.ds(t0, SUB)], i0_v)
            pltpu.sync_copy(p1_hbm.at[pl.ds(t0, SUB)], i1_v)
            pltpu.sync_copy(gs0_hbm.at[pl.ds(t0, SUB)], ga_v)
            pltpu.sync_copy(gs1_hbm.at[pl.ds(t0, SUB)], gb_v)
            pltpu.async_copy(os_hbm.at[i0_v], a_v, sem).wait()
            pltpu.async_copy(os_hbm.at[i1_v], b_v, sem).wait()

            def row(r, c2):
                s0 = ga_v[r, :]
                s1 = gb_v[r, :]
                for v in range(D // 16):
                    sl = pl.ds(v * 16, 16)
                    a_v[r, sl] = s0 * a_v[r, sl] + s1 * b_v[r, sl]
                return c2

            lax.fori_loop(0, SUB, row, 0)
            pltpu.sync_copy(a_v, y_hbm.at[pl.ds(t0, SUB)])
            return carry

        lax.fori_loop(0, NSUB, sub, 0)

    return k(outs, pos0, pos1, gs0, gs1)


def kernel(x, w_gate, W1, W2):
    bsz, length, d = x.shape
    xf = x.reshape(-1, d)
    g0, g1, e0, e1, r0, r1, cnt, gs0, gs1 = _route(xf, w_gate)
    pos0, pos1, bexp, bval, sf, so, nx, ns = _finalize(cnt, e0, e1, r0, r1)
    pos0 = pos0.reshape(-1)
    pos1 = pos1.reshape(-1)
    ei = _dispatch(xf, pos0, pos1)
    outs = _ffn(bexp.reshape(-1), bval.reshape(-1), sf.reshape(-1),
                so.reshape(-1), nx.reshape(-1), ns.reshape(-1), ei, W1, W2)
    y = _combine(outs, pos0, pos1, gs0, gs1)
    return y.reshape(bsz, length, d)


# speedup vs baseline: 1.3410x; 1.0087x over previous
import functools

import jax
import jax.numpy as jnp
from jax import lax
from jax.experimental import pallas as pl
from jax.experimental.pallas import tpu as pltpu
from jax.experimental.pallas import tpu_sc as plsc

E = 16
K = 2
NT = 4096
D = 1024
H = 2048
T = 128
P = NT * K + E * T
NB = P // T
CHUNK = 512
NC, NS = 2, 16
NW = NC * NS
TPW = NT // NW
SUB = 32
NSUB = TPW // SUB


def _route_body(x_ref, wg_ref, g0_ref, g1_ref, e0_ref, e1_ref,
                r0_ref, r1_ref, cnt_ref, gs0_ref, gs1_ref, run_ref):
    step = pl.program_id(0)

    @pl.when(step == 0)
    def _():
        run_ref[...] = jnp.zeros_like(run_ref)

    logits = jnp.dot(x_ref[...], wg_ref[...],
                     preferred_element_type=jnp.float32)
    m = jnp.max(logits, axis=-1, keepdims=True)
    ex = jnp.exp(logits - m)
    probs = ex / jnp.sum(ex, axis=-1, keepdims=True)

    iota_e = lax.broadcasted_iota(jnp.int32, (CHUNK, E), 1)
    m1 = jnp.max(probs, axis=-1, keepdims=True)
    a1 = jnp.min(jnp.where(probs == m1, iota_e, E), axis=-1, keepdims=True)
    probs2 = jnp.where(iota_e == a1, -1.0, probs)
    m2 = jnp.max(probs2, axis=-1, keepdims=True)
    a2 = jnp.min(jnp.where(probs2 == m2, iota_e, E), axis=-1, keepdims=True)

    g0_ref[...] = m1
    g1_ref[...] = m2
    e0_ref[...] = a1
    e1_ref[...] = a2
    gs0_ref[...] = jnp.broadcast_to(m1, (CHUNK, E))
    gs1_ref[...] = jnp.broadcast_to(m2, (CHUNK, E))

    oh1 = (iota_e == a1).astype(jnp.float32)
    oh2 = (iota_e == a2).astype(jnp.float32)

    rr = lax.broadcasted_iota(jnp.int32, (CHUNK, CHUNK), 0)
    cc = lax.broadcasted_iota(jnp.int32, (CHUNK, CHUNK), 1)
    lt = (rr > cc).astype(jnp.float32)
    run = run_ref[0:1, :]
    cs1 = jnp.sum(oh1, axis=0, keepdims=True)
    ranks1 = jnp.dot(lt, oh1, preferred_element_type=jnp.float32) + run
    ranks2 = (jnp.dot(lt, oh2, preferred_element_type=jnp.float32)
              + run + cs1)
    r0_ref[...] = jnp.sum(ranks1 * oh1, axis=-1,
                          keepdims=True).astype(jnp.int32)
    r1_ref[...] = jnp.sum(ranks2 * oh2, axis=-1,
                          keepdims=True).astype(jnp.int32)

    run_ref[0:1, :] = run + cs1 + jnp.sum(oh2, axis=0, keepdims=True)
    cnt_ref[...] = run_ref[...]


def _route(xf, w_gate):
    n_steps = NT // CHUNK
    col_i = jax.ShapeDtypeStruct((NT, 1), jnp.int32)
    col_f = jax.ShapeDtypeStruct((NT, 1), jnp.float32)
    spec_col = pl.BlockSpec((CHUNK, 1), lambda i: (i, 0))
    return pl.pallas_call(
        _route_body,
        grid=(n_steps,),
        in_specs=[
            pl.BlockSpec((CHUNK, D), lambda i: (i, 0)),
            pl.BlockSpec((D, E), lambda i: (0, 0)),
        ],
        out_specs=[spec_col, spec_col, spec_col, spec_col, spec_col, spec_col,
                   pl.BlockSpec((8, E), lambda i: (0, 0)),
                   pl.BlockSpec((CHUNK, E), lambda i: (i, 0)),
                   pl.BlockSpec((CHUNK, E), lambda i: (i, 0))],
        out_shape=[col_f, col_f, col_i, col_i, col_i, col_i,
                   jax.ShapeDtypeStruct((8, E), jnp.float32),
                   jax.ShapeDtypeStruct((NT, E), jnp.float32),
                   jax.ShapeDtypeStruct((NT, E), jnp.float32)],
        scratch_shapes=[pltpu.VMEM((8, E), jnp.float32)],
    )(xf, w_gate)


def _shift_lanes(a, k):
    return jnp.concatenate([jnp.zeros((1, k), a.dtype), a[:, :E - k]], axis=1)


def _shift_rows(a, k):
    return jnp.concatenate([jnp.zeros((k, 1), a.dtype), a[:a.shape[0] - k, :]],
                           axis=0)


def _finalize_body(cnt_ref, e0_ref, e1_ref, r0_ref, r1_ref,
                   p0_ref, p1_ref, be_ref, bv_ref, sf_ref, so_ref,
                   nx_ref, nx2_ref, ns_ref):
    c = cnt_ref[0:1, :].astype(jnp.int32)
    padded = ((c + T - 1) // T) * T
    s = padded
    for k in (1, 2, 4, 8):
        s = s + _shift_lanes(s, k)
    offs = _shift_lanes(s, 1)
    total = jnp.sum(padded)

    def lookup(e_col):
        cmp = e_col == lax.broadcasted_iota(jnp.int32, (NT, E), 1)
        return jnp.sum(jnp.where(cmp, offs, 0), axis=1, keepdims=True)

    p0_ref[...] = r0_ref[...] + lookup(e0_ref[...])
    p1_ref[...] = r1_ref[...] + lookup(e1_ref[...])

    iota_be = lax.broadcasted_iota(jnp.int32, (NB, E), 1)
    bT = lax.broadcasted_iota(jnp.int32, (NB, E), 0) * T
    be = jnp.sum((offs <= bT).astype(jnp.int32), axis=1, keepdims=True) - 1
    be = jnp.clip(be, 0, E - 1)
    bv = (lax.broadcasted_iota(jnp.int32, (NB, 1), 0) * T
          < total).astype(jnp.int32)
    be_ref[...] = be
    bv_ref[...] = bv

    prev = jnp.concatenate([jnp.full((1, 1), -1, jnp.int32), be[:-1, :]],
                           axis=0)
    sf = ((be != prev) & (bv != 0)).astype(jnp.int32)
    sf_ref[...] = sf
    cum = sf
    for k in (1, 2, 4, 8, 16, 32, 64):
        if k < NB:
            cum = cum + _shift_rows(cum, k)
    so_ref[...] = cum - 1
    ns_ref[...] = jnp.sum(sf) * jnp.ones((1, 1), jnp.int32)

    cand = (iota_be > be) & (padded > 0)
    nx = jnp.min(jnp.where(cand, iota_be, E - 1), axis=1, keepdims=True)
    nx_ref[...] = nx
    cand2 = (iota_be > nx) & (padded > 0)
    nx2_ref[...] = jnp.min(jnp.where(cand2, iota_be, E - 1), axis=1,
                           keepdims=True)


def _finalize(cnt, e0, e1, r0, r1):
    col_i = jax.ShapeDtypeStruct((NT, 1), jnp.int32)
    blk_i = jax.ShapeDtypeStruct((NB, 1), jnp.int32)
    one_i = jax.ShapeDtypeStruct((1, 1), jnp.int32)
    return pl.pallas_call(
        _finalize_body,
        out_shape=[col_i, col_i, blk_i, blk_i, blk_i, blk_i, blk_i, blk_i,
                   one_i],
    )(cnt, e0, e1, r0, r1)


def _dispatch(xf, pos0, pos1):
    mesh = plsc.VectorSubcoreMesh(core_axis_name="c", subcore_axis_name="s")

    @functools.partial(
        pl.kernel,
        mesh=mesh,
        out_type=jax.ShapeDtypeStruct((P, D), jnp.float32),
        scratch_types=[
            pltpu.VMEM((SUB, D), jnp.float32),
            pltpu.VMEM((SUB,), jnp.int32),
            pltpu.VMEM((SUB,), jnp.int32),
            pltpu.SemaphoreType.DMA,
        ],
    )
    def k(x_hbm, p0_hbm, p1_hbm, out_hbm, rows_v, i0_v, i1_v, sem):
        wid = lax.axis_index("s") * NC + lax.axis_index("c")

        def sub(j, carry):
            t0 = wid * TPW + j * SUB
            pltpu.sync_copy(x_hbm.at[pl.ds(t0, SUB)], rows_v)
            pltpu.sync_copy(p0_hbm.at[pl.ds(t0, SUB)], i0_v)
            pltpu.sync_copy(p1_hbm.at[pl.ds(t0, SUB)], i1_v)
            pltpu.async_copy(rows_v, out_hbm.at[i0_v], sem).wait()
            pltpu.async_copy(rows_v, out_hbm.at[i1_v], sem).wait()
            return carry

        lax.fori_loop(0, NSUB, sub, 0)

    return k(xf, pos0, pos1)


def _ffn_body(be_ref, bv_ref, sf_ref, so_ref, nx_ref, nx2_ref, ns_ref,
              x_ref, w1_hbm, w2_hbm, o_ref, w1b, w2b, s1, s2):
    b = pl.program_id(0)
    s = so_ref[b]
    slot = lax.rem(s, 3)

    def issue(e, sl):
        pltpu.make_async_copy(w1_hbm.at[e], w1b.at[sl], s1.at[sl]).start()
        pltpu.make_async_copy(w2_hbm.at[e], w2b.at[sl], s2.at[sl]).start()

    @pl.when(b == 0)
    def _():
        issue(be_ref[0], 0)

        @pl.when(ns_ref[0] > 1)
        def _():
            issue(nx_ref[0], 1)

    @pl.when(sf_ref[b] != 0)
    def _():
        pltpu.make_async_copy(w1_hbm.at[be_ref[b]], w1b.at[slot],
                              s1.at[slot]).wait()
        pltpu.make_async_copy(w2_hbm.at[be_ref[b]], w2b.at[slot],
                              s2.at[slot]).wait()

        @pl.when(s + 2 < ns_ref[0])
        def _():
            issue(nx2_ref[b], lax.rem(s + 2, 3))

    @pl.when(bv_ref[b] != 0)
    def _():
        h = jnp.dot(x_ref[...], w1b[slot], preferred_element_type=jnp.float32)
        h = jax.nn.gelu(h)
        o_ref[...] = jnp.dot(h, w2b[slot], preferred_element_type=jnp.float32)

    @pl.when(bv_ref[b] == 0)
    def _():
        o_ref[...] = jnp.zeros_like(o_ref)


def _ffn(bexp, bval, sf, so, nx, nx2, ns, ei, W1, W2):
    grid_spec = pltpu.PrefetchScalarGridSpec(
        num_scalar_prefetch=7,
        grid=(NB,),
        in_specs=[
            pl.BlockSpec((T, D), lambda b, *_: (b, 0)),
            pl.BlockSpec(memory_space=pl.ANY),
            pl.BlockSpec(memory_space=pl.ANY),
        ],
        out_specs=pl.BlockSpec((T, D), lambda b, *_: (b, 0)),
        scratch_shapes=[
            pltpu.VMEM((3, D, H), jnp.float32),
            pltpu.VMEM((3, H, D), jnp.float32),
            pltpu.SemaphoreType.DMA((3,)),
            pltpu.SemaphoreType.DMA((3,)),
        ],
    )
    return pl.pallas_call(
        _ffn_body,
        grid_spec=grid_spec,
        out_shape=jax.ShapeDtypeStruct((P, D), jnp.float32),
    )(bexp, bval, sf, so, nx, nx2, ns, ei, W1, W2)


def _combine(outs, pos0, pos1, gs0, gs1):
    mesh = plsc.VectorSubcoreMesh(core_axis_name="c", subcore_axis_name="s")

    @functools.partial(
        pl.kernel,
        mesh=mesh,
        out_type=jax.ShapeDtypeStruct((NT, D), jnp.float32),
        scratch_types=[
            pltpu.VMEM((SUB, D), jnp.float32),
            pltpu.VMEM((SUB, D), jnp.float32),
            pltpu.VMEM((SUB, E), jnp.float32),
            pltpu.VMEM((SUB, E), jnp.float32),
            pltpu.VMEM((SUB,), jnp.int32),
            pltpu.VMEM((SUB,), jnp.int32),
            pltpu.SemaphoreType.DMA,
        ],
    )
    def k(os_hbm, p0_hbm, p1_hbm, gs0_hbm, gs1_hbm, y_hbm,
          a_v, b_v, ga_v, gb_v, i0_v, i1_v, sem):
        wid = lax.axis_index("s") * NC + lax.axis_index("c")

        def sub(j, carry):
            t0 = wid * TPW + j * SUB
            pltpu.sync_copy(p0_hbm.at[pl.ds(t0, SUB)], i0_v)
            pltpu.sync_copy(p1_hbm.at[pl.ds(t0, SUB)], i1_v)
            pltpu.sync_copy(gs0_hbm.at[pl.ds(t0, SUB)], ga_v)
            pltpu.sync_copy(gs1_hbm.at[pl.ds(t0, SUB)], gb_v)
            pltpu.async_copy(os_hbm.at[i0_v], a_v, sem).wait()
            pltpu.async_copy(os_hbm.at[i1_v], b_v, sem).wait()

            def row(r, c2):
                s0 = ga_v[r, :]
                s1 = gb_v[r, :]
                for v in range(D // 16):
                    sl = pl.ds(v * 16, 16)
                    a_v[r, sl] = s0 * a_v[r, sl] + s1 * b_v[r, sl]
                return c2

            lax.fori_loop(0, SUB, row, 0)
            pltpu.sync_copy(a_v, y_hbm.at[pl.ds(t0, SUB)])
            return carry

        lax.fori_loop(0, NSUB, sub, 0)

    return k(outs, pos0, pos1, gs0, gs1)


def kernel(x, w_gate, W1, W2):
    bsz, length, d = x.shape
    xf = x.reshape(-1, d)
    g0, g1, e0, e1, r0, r1, cnt, gs0, gs1 = _route(xf, w_gate)
    (pos0, pos1, bexp, bval, sf, so,
     nx, nx2, ns) = _finalize(cnt, e0, e1, r0, r1)
    pos0 = pos0.reshape(-1)
    pos1 = pos1.reshape(-1)
    ei = _dispatch(xf, pos0, pos1)
    outs = _ffn(bexp.reshape(-1), bval.reshape(-1), sf.reshape(-1),
                so.reshape(-1), nx.reshape(-1), nx2.reshape(-1),
                ns.reshape(-1), ei, W1, W2)
    y = _combine(outs, pos0, pos1, gs0, gs1)
    return y.reshape(bsz, length, d)
